# Initial kernel scaffold; baseline (speedup 1.0000x reference)
import jax
import jax.numpy as jnp
from jax.experimental import pallas as pl


def kernel(input_ids, table):
    raise NotImplementedError("write your pallas kernel here")


# speedup vs baseline: 2.1533x; 2.1533x over previous
import functools

import jax
import jax.numpy as jnp
from jax import lax
from jax.experimental import pallas as pl
from jax.experimental.pallas import tpu as pltpu
from jax.experimental.pallas import tpu_sc as plsc

_B, _S, _H, _D = 1024, 200, 8, 32
_NPH = 100000
_NB = 64
_NBC = _B // _NB
_IPW = (_S // 2) // 2
_NBUF = 2

_mesh = plsc.VectorSubcoreMesh(core_axis_name="c", subcore_axis_name="s")


@functools.partial(
    pl.kernel,
    out_type=jax.ShapeDtypeStruct((_S, _B, _H, _D), jnp.float32),
    mesh=_mesh,
    compiler_params=pltpu.CompilerParams(use_tc_tiling_on_sc=False),
    scratch_types=[
        pltpu.VMEM((_NBUF, _NB, 16), jnp.int32),
        pltpu.VMEM((_NBUF, _NB, 16, _D), jnp.float32),
        pltpu.SemaphoreType.DMA,
        pltpu.SemaphoreType.DMA,
        pltpu.SemaphoreType.DMA,
        pltpu.SemaphoreType.DMA,
        pltpu.SemaphoreType.DMA,
        pltpu.SemaphoreType.DMA,
    ],
)
def _mhe_kernel(ids_hbm, table_hbm, out_hbm, raw_v, rows_v,
                sem_i0, sem_i1, sem_g0, sem_g1, sem_w0, sem_w1):
    wid = lax.axis_index("s") * 2 + lax.axis_index("c")
    bc = wid % _NBC
    b0 = bc * _NB
    p0 = (wid // _NBC) * _IPW

    sem_i = (sem_i0, sem_i1)
    sem_g = (sem_g0, sem_g1)
    sem_w = (sem_w0, sem_w1)

    iota = lax.iota(jnp.int32, 16)
    offv = (iota & 7) * _NPH

    def idx_copy(item, slot):
        p = p0 + item
        return pltpu.make_async_copy(
            ids_hbm.at[pl.ds(b0, _NB), pl.ds(p * 16, 16)],
            raw_v.at[slot], sem_i[slot])

    def write_copy(item, slot, sl):
        s = (p0 + item) * 2 + sl
        return pltpu.make_async_copy(
            rows_v.at[slot, :, pl.ds(sl * _H, _H), :],
            out_hbm.at[s, pl.ds(b0, _NB)], sem_w[slot])

    def gather_drain(item, slot, sl):
        s = (p0 + item) * 2 + sl
        return pltpu.make_async_copy(
            out_hbm.at[s, pl.ds(b0, _NB)],
            rows_v.at[slot, :, pl.ds(sl * _H, _H), :], sem_g[slot])

    idx_copy(0, 0).start()
    idx_copy(1, 1).start()

    @pl.loop(0, _IPW, step=_NBUF)
    def _item_pair(i0):
        for slot in range(_NBUF):
            it = i0 + slot
            idx_copy(it, slot).wait()
            for b in range(_NB):
                raw_v[slot, b, :] = raw_v[slot, b, :] + offv

            @pl.when(it >= _NBUF)
            def _():
                write_copy(it - _NBUF, slot, 0).wait()
                write_copy(it - _NBUF, slot, 1).wait()

            for b in range(_NB):
                pltpu.make_async_copy(
                    table_hbm.at[raw_v.at[slot, b]],
                    rows_v.at[slot, b], sem_g[slot]).start()
            gather_drain(it, slot, 0).wait()
            gather_drain(it, slot, 1).wait()

            @pl.when(it + _NBUF < _IPW)
            def _():
                idx_copy(it + _NBUF, slot).start()

            write_copy(it, slot, 0).start()
            write_copy(it, slot, 1).start()

    for slot in range(_NBUF):
        write_copy(_IPW - _NBUF + slot, slot, 0).wait()
        write_copy(_IPW - _NBUF + slot, slot, 1).wait()


def kernel(input_ids, table):
    ids2 = input_ids.reshape(_B, _S * _H)
    return _mhe_kernel(ids2, table)


# speedup vs baseline: 2.1997x; 1.0215x over previous
import functools

import jax
import jax.numpy as jnp
from jax import lax
from jax.experimental import pallas as pl
from jax.experimental.pallas import tpu as pltpu
from jax.experimental.pallas import tpu_sc as plsc

_B, _S, _H, _D = 1024, 200, 8, 32
_NPH = 100000
_NB = 32
_NBC = _B // _NB
_IPW = _S // 2
_NSL = _NB * _H
_NBUF = 4

_mesh = plsc.VectorSubcoreMesh(core_axis_name="c", subcore_axis_name="s")


@functools.partial(
    pl.kernel,
    out_type=jax.ShapeDtypeStruct((_S, _B * _H, _D), jnp.float32),
    mesh=_mesh,
    compiler_params=pltpu.CompilerParams(
        use_tc_tiling_on_sc=False, needs_layout_passes=False),
    scratch_types=[
        pltpu.VMEM((_NBUF, _NB, 16), jnp.int32),
        pltpu.VMEM((_NBUF, 2, 2, 128), jnp.int32),
        pltpu.VMEM((_NBUF, 2, _NSL, _D), jnp.float32),
        pltpu.SemaphoreType.DMA,
        pltpu.SemaphoreType.DMA,
        pltpu.SemaphoreType.DMA,
        pltpu.SemaphoreType.DMA,
        pltpu.SemaphoreType.DMA,
        pltpu.SemaphoreType.DMA,
        pltpu.SemaphoreType.DMA,
        pltpu.SemaphoreType.DMA,
        pltpu.SemaphoreType.DMA,
        pltpu.SemaphoreType.DMA,
        pltpu.SemaphoreType.DMA,
        pltpu.SemaphoreType.DMA,
    ],
)
def _mhe_kernel(ids_hbm, table_hbm, out_hbm, raw_v, gidx_v, rows_v,
                sem_i0, sem_i1, sem_i2, sem_i3,
                sem_g0, sem_g1, sem_g2, sem_g3,
                sem_w0, sem_w1, sem_w2, sem_w3):
    wid = lax.axis_index("s") * 2 + lax.axis_index("c")
    b0 = wid * _NB
    o0 = wid * _NSL

    sem_i = (sem_i0, sem_i1, sem_i2, sem_i3)
    sem_g = (sem_g0, sem_g1, sem_g2, sem_g3)
    sem_w = (sem_w0, sem_w1, sem_w2, sem_w3)

    iota = lax.iota(jnp.int32, 16)
    rv = iota >> 3
    cv = iota & 7
    cv8 = cv + 8
    offv = cv * _NPH

    def idx_copy(item, slot):
        return pltpu.make_async_copy(
            ids_hbm.at[pl.ds(b0, _NB), pl.ds(item * 16, 16)],
            raw_v.at[slot], sem_i[slot])

    def write_copy(item, slot, sl):
        return pltpu.make_async_copy(
            rows_v.at[slot, sl],
            out_hbm.at[item * 2 + sl, pl.ds(o0, _NSL)], sem_w[slot])

    def gather_drain(item, slot, sl):
        return pltpu.make_async_copy(
            out_hbm.at[item * 2 + sl, pl.ds(o0, _NSL)],
            rows_v.at[slot, sl], sem_g[slot])

    def stage(slot):
        rowv = rv
        for g in range(2):
            for k in range(8):
                v0 = plsc.load_gather(raw_v.at[slot], [rowv, cv]) + offv
                gidx_v[slot, 0, g, pl.ds(16 * k, 16)] = v0
                v1 = plsc.load_gather(raw_v.at[slot], [rowv, cv8]) + offv
                gidx_v[slot, 1, g, pl.ds(16 * k, 16)] = v1
                rowv = rowv + 2

    for slot in range(_NBUF):
        idx_copy(slot, slot).start()

    @pl.loop(0, _IPW, step=_NBUF)
    def _item_quad(i0):
        for slot in range(_NBUF):
            it = i0 + slot
            sp = (slot - 1) % _NBUF
            idx_copy(it, slot).wait()
            stage(slot)

            @pl.when(it + _NBUF < _IPW)
            def _():
                idx_copy(it + _NBUF, slot).start()

            @pl.when(it >= _NBUF)
            def _():
                write_copy(it - _NBUF, slot, 0).wait()
                write_copy(it - _NBUF, slot, 1).wait()

            for sl in range(2):
                for g in range(2):
                    pltpu.make_async_copy(
                        table_hbm.at[gidx_v.at[slot, sl, g]],
                        rows_v.at[slot, sl, pl.ds(g * 128, 128)],
                        sem_g[slot]).start()

            @pl.when(it >= 1)
            def _():
                gather_drain(it - 1, sp, 0).wait()
                gather_drain(it - 1, sp, 1).wait()
                write_copy(it - 1, sp, 0).start()
                write_copy(it - 1, sp, 1).start()

    last = _IPW - 1
    lslot = last % _NBUF
    gather_drain(last, lslot, 0).wait()
    gather_drain(last, lslot, 1).wait()
    write_copy(last, lslot, 0).start()
    write_copy(last, lslot, 1).start()
    for k in range(_NBUF):
        it = _IPW - _NBUF + k
        write_copy(it, it % _NBUF, 0).wait()
        write_copy(it, it % _NBUF, 1).wait()


def kernel(input_ids, table):
    ids2 = input_ids.reshape(_B, _S * _H)
    out = _mhe_kernel(ids2, table)
    return out.reshape(_S, _B, _H, _D)
